# baseline (speedup 1.0000x reference)
import functools

import jax
import jax.numpy as jnp
from jax import lax
from jax.experimental import pallas as pl
from jax.experimental.pallas import tpu as pltpu
from jax.experimental.pallas import tpu_sc as plsc

B = 8
NQ = 5000
NT = 1000
L = 16

PRED_CHUNK = (NQ + L - 1) // L
TGT_CHUNK = (NT + L - 1) // L
PAD_N = PRED_CHUNK * L

RADIX_BITS = 8
NBINS = 1 << RADIX_BITS
NPASS = 4


def _sortable_key(area):
    a = area + 0.0
    u = plsc.bitcast(a, jnp.int32)
    s = lax.shift_right_arithmetic(u, 31)
    m = u ^ (s | jnp.int32(-2147483648))
    return ~m


def _radix_argsort(boxes_v, ka, kb, va, vb, hist, n, chunk):
    lane = lax.iota(jnp.int32, L)
    zeros = jnp.zeros((L,), jnp.int32)
    ones = jnp.ones((L,), jnp.int32)

    def build(i, _):
        e = lane + i * L
        b4 = e * 4
        x1 = plsc.load_gather(boxes_v, [b4])
        y1 = plsc.load_gather(boxes_v, [b4 + 1])
        x2 = plsc.load_gather(boxes_v, [b4 + 2])
        y2 = plsc.load_gather(boxes_v, [b4 + 3])
        k = _sortable_key((x2 - x1) * (y2 - y1))
        k = jnp.where(e < n, k, jnp.int32(-1))
        ka[pl.ds(i * L, L)] = k
        va[pl.ds(i * L, L)] = e
        return 0

    lax.fori_loop(0, chunk, build, 0, unroll=2)

    src = (ka, va)
    dst = (kb, vb)
    for p in range(NPASS):
        shift = p * RADIX_BITS
        ks, vs = src
        kd, vd = dst

        def clear(j, _):
            hist[pl.ds(j * L, L)] = zeros
            return 0

        lax.fori_loop(0, NBINS, clear, 0, unroll=4)

        def histo(i, _):
            e = lane * chunk + i
            k = plsc.load_gather(ks, [e])
            d = lax.shift_right_logical(k, shift) & (NBINS - 1)
            plsc.addupdate_scatter(hist, [d * L + lane], ones)
            return 0

        lax.fori_loop(0, chunk, histo, 0, unroll=2)

        def scan(j, carry):
            v = hist[pl.ds(j * L, L)]
            inc = plsc.cumsum(v)
            hist[pl.ds(j * L, L)] = inc - v + carry
            return carry + jnp.sum(v)

        lax.fori_loop(0, NBINS, scan, jnp.int32(0), unroll=2)

        def permute(i, _):
            e = lane * chunk + i
            k = plsc.load_gather(ks, [e])
            v = plsc.load_gather(vs, [e])
            d = lax.shift_right_logical(k, shift) & (NBINS - 1)
            h = d * L + lane
            o = plsc.load_gather(hist, [h])
            plsc.store_scatter(kd, [o], k)
            plsc.store_scatter(vd, [o], v)
            plsc.store_scatter(hist, [h], o + 1)
            return 0

        lax.fori_loop(0, chunk, permute, 0, unroll=2)

        src, dst = dst, src


def _matcher_body(pred_hbm, tgt_hbm, out_pred, out_tgt,
                  boxes_v, ka, kb, va, vb, hist):
    c = lax.axis_index("c")
    s = lax.axis_index("s")
    is_pred = s < 4
    is_tgt = (s >= 4) & (s < 8)
    pred_row = c * 4 + s
    tgt_row = c * 4 + (s - 4)

    @pl.when(is_pred)
    def _():
        pltpu.sync_copy(pred_hbm.at[pred_row], boxes_v.at[pl.ds(0, NQ * 4)])
        _radix_argsort(boxes_v, ka, kb, va, vb, hist, NQ, PRED_CHUNK)
        pltpu.sync_copy(va.at[pl.ds(0, NT)], out_pred.at[pred_row])

    @pl.when(is_tgt)
    def _():
        pltpu.sync_copy(tgt_hbm.at[tgt_row], boxes_v.at[pl.ds(0, NT * 4)])
        _radix_argsort(boxes_v, ka, kb, va, vb, hist, NT, TGT_CHUNK)
        pltpu.sync_copy(va.at[pl.ds(0, NT)], out_tgt.at[tgt_row])


@jax.jit
def _match(pred_boxes, target_boxes):
    run = functools.partial(
        pl.kernel,
        out_type=[
            jax.ShapeDtypeStruct((B, NT), jnp.int32),
            jax.ShapeDtypeStruct((B, NT), jnp.int32),
        ],
        mesh=plsc.VectorSubcoreMesh(core_axis_name="c", subcore_axis_name="s"),
        compiler_params=pltpu.CompilerParams(
            needs_layout_passes=False, use_tc_tiling_on_sc=False),
        scratch_types=[
            pltpu.VMEM((PAD_N * 4,), jnp.float32),
            pltpu.VMEM((PAD_N,), jnp.int32),
            pltpu.VMEM((PAD_N,), jnp.int32),
            pltpu.VMEM((PAD_N,), jnp.int32),
            pltpu.VMEM((PAD_N,), jnp.int32),
            pltpu.VMEM((NBINS * L,), jnp.int32),
        ],
    )(_matcher_body)
    return run(pred_boxes.reshape(B, NQ * 4), target_boxes.reshape(B, NT * 4))


def kernel(logits, pred_boxes, target_boxes, class_labels):
    del logits, class_labels
    matched_pred, matched_tgt = _match(pred_boxes, target_boxes)
    return (matched_pred, matched_tgt)
